# Initial kernel scaffold; baseline (speedup 1.0000x reference)
import functools

import jax
import jax.numpy as jnp
from jax import lax
from jax.experimental import pallas as pl
from jax.experimental.pallas import tpu as pltpu
from jax.experimental.pallas import tpu_sc as plsc

N = 10000
E = 320000
D = 128
L = 16
NC = 2
NS = 16
NPT = 640
NPAD = NS * NPT
E_HALF = E // NC
C = 2000
R = 128
CBUF = ((C + R - 1) // R) * R
KD = D // L


def _sc_body(x_hbm, edge_hbm, out_hbm, acc, src_c, dst_c, msrc, mloc, rows, sem):
    c = lax.axis_index("c")
    s = lax.axis_index("s")
    lo = s * NPT
    neg = jnp.full((L,), -jnp.inf, dtype=jnp.float32)
    zero_idx = jnp.zeros((L,), dtype=jnp.int32)
    dummy = jnp.full((L,), NPT, dtype=jnp.int32)

    def init_r(r, _):
        for k in range(KD):
            acc[r, pl.ds(k * L, L)] = neg
        return 0
    lax.fori_loop(0, NPT + 1, init_r, 0)

    def init_m(i, _):
        msrc[pl.ds(i * L, L)] = zero_idx
        return 0
    lax.fori_loop(0, CBUF // L, init_m, 0)

    def chunk_body(ch, _):
        off = c * E_HALF + ch * C
        pltpu.sync_copy(edge_hbm.at[0, pl.ds(off, C)], src_c)
        pltpu.sync_copy(edge_hbm.at[1, pl.ds(off, C)], dst_c)

        def fill_i(i, _):
            mloc[pl.ds(i * L, L)] = dummy
            return 0
        lax.fori_loop(0, CBUF // L, fill_i, 0)

        def scan_i(i, cnt):
            d = dst_c[pl.ds(i * L, L)]
            sv = src_c[pl.ds(i * L, L)]
            locv = d - lo
            m = (locv >= 0) & (locv < NPT)
            plsc.store_compressed(msrc.at[pl.ds(cnt, L)], sv, mask=m)
            plsc.store_compressed(mloc.at[pl.ds(cnt, L)], locv, mask=m)
            return cnt + jnp.sum(m.astype(jnp.int32))
        cnt = lax.fori_loop(0, C // L, scan_i, 0)

        nb = (cnt + R - 1) // R

        def batch_b(bi, _):
            pltpu.async_copy(
                x_hbm.at[msrc.at[pl.ds(bi * R, R)]], rows, sem).wait()

            def edge_j(j, _):
                loc = mloc[bi * R + j]
                for k in range(KD):
                    a = acc[loc, pl.ds(k * L, L)]
                    v = rows[j, pl.ds(k * L, L)]
                    acc[loc, pl.ds(k * L, L)] = jnp.maximum(a, v)
                return 0
            lax.fori_loop(0, R, edge_j, 0)
            return 0
        lax.fori_loop(0, nb, batch_b, 0)
        return 0

    lax.fori_loop(0, E_HALF // C, chunk_body, 0)

    pltpu.sync_copy(acc.at[pl.ds(0, NPT), :], out_hbm.at[c, pl.ds(lo, NPT), :])


@jax.jit
def _sc_segmax(x, edge_index):
    mesh = plsc.VectorSubcoreMesh(core_axis_name="c", subcore_axis_name="s")
    return pl.kernel(
        _sc_body,
        out_type=jax.ShapeDtypeStruct((NC, NPAD, D), jnp.float32),
        mesh=mesh,
        scratch_types=[
            pltpu.VMEM((NPT + 1, D), jnp.float32),
            pltpu.VMEM((C,), jnp.int32),
            pltpu.VMEM((C,), jnp.int32),
            pltpu.VMEM((CBUF,), jnp.int32),
            pltpu.VMEM((CBUF,), jnp.int32),
            pltpu.VMEM((R, D), jnp.float32),
            pltpu.SemaphoreType.DMA,
        ],
    )(x, edge_index)


def _tc_body(x_ref, p_ref, w1_ref, w2_ref, b_ref, o_ref):
    m = jnp.maximum(p_ref[0], p_ref[1])
    x = x_ref[...]
    agg = jnp.where(m == -jnp.inf, 0.0, m - x)
    h = (jnp.dot(x, w1_ref[...], preferred_element_type=jnp.float32)
         + jnp.dot(agg, w2_ref[...], preferred_element_type=jnp.float32)
         + b_ref[...])
    o_ref[...] = jnp.maximum(h, 0.0)


@jax.jit
def _tc_mlp(x, part, w1, w2, b2d):
    blk = 2000
    grid = N // blk
    return pl.pallas_call(
        _tc_body,
        grid=(grid,),
        in_specs=[
            pl.BlockSpec((blk, D), lambda i: (i, 0)),
            pl.BlockSpec((NC, blk, D), lambda i: (0, i, 0)),
            pl.BlockSpec((D, D), lambda i: (0, 0)),
            pl.BlockSpec((D, D), lambda i: (0, 0)),
            pl.BlockSpec((1, D), lambda i: (0, 0)),
        ],
        out_specs=pl.BlockSpec((blk, D), lambda i: (i, 0)),
        out_shape=jax.ShapeDtypeStruct((N, D), jnp.float32),
    )(x, part, w1, w2, b2d)


def kernel(x, edge_index, W, b):
    part = _sc_segmax(x, edge_index)
    return _tc_mlp(x, part, W[:D], W[D:], b.reshape(1, D))


# speedup vs baseline: 1.3228x; 1.3228x over previous
import functools

import jax
import jax.numpy as jnp
from jax import lax
from jax.experimental import pallas as pl
from jax.experimental.pallas import tpu as pltpu
from jax.experimental.pallas import tpu_sc as plsc

N = 10000
E = 320000
D = 128
L = 16
NC = 2
NS = 16
NPT = 640
NPAD = NS * NPT
E_HALF = E // NC
C = 3200
R = 128
CBUF = ((C + R - 1) // R) * R
KD = D // L


def _sc_body(x_hbm, edge_hbm, out_hbm, acc, edge_c, msrc, mloc, rows, sem):
    c = lax.axis_index("c")
    s = lax.axis_index("s")
    lo = s * NPT
    neg = jnp.full((L,), -jnp.inf, dtype=jnp.float32)
    zero_idx = jnp.zeros((L,), dtype=jnp.int32)
    dummy = jnp.full((L,), NPT, dtype=jnp.int32)

    def init_r(r, _):
        for k in range(KD):
            acc[r, pl.ds(k * L, L)] = neg
        return 0
    lax.fori_loop(0, NPT + 1, init_r, 0)

    def init_m(i, _):
        msrc[pl.ds(i * L, L)] = zero_idx
        return 0
    lax.fori_loop(0, CBUF // L, init_m, 0)

    def chunk_body(ch, _):
        off = c * E_HALF + ch * C
        pltpu.sync_copy(edge_hbm.at[:, pl.ds(off, C)], edge_c)

        def fill_i(i, _):
            mloc[pl.ds(i * L, L)] = dummy
            return 0
        lax.fori_loop(0, CBUF // L, fill_i, 0)

        lov = jnp.broadcast_to(lo, (L,))
        zv = jnp.zeros((L,), dtype=jnp.int32)
        nptv = jnp.full((L,), NPT, dtype=jnp.int32)
        onev = jnp.full((L,), 1, dtype=jnp.int32)

        def scan_i(i, cnt):
            d = edge_c[1, pl.ds(i * L, L)]
            sv = edge_c[0, pl.ds(i * L, L)]
            locv = d - lov
            m = (locv >= zv) & (locv < nptv)
            mi = jnp.where(m, onev, zv)
            pos = jnp.broadcast_to(cnt, (L,)) + plsc.cumsum(mi) - onev
            plsc.store_scatter(msrc, [pos], sv, mask=m)
            plsc.store_scatter(mloc, [pos], locv, mask=m)
            return cnt + jnp.sum(mi)
        cnt = lax.fori_loop(0, C // L, scan_i, 0)

        nb = (cnt + R - 1) // R

        def batch_b(bi, _):
            pltpu.async_copy(
                x_hbm.at[msrc.at[pl.ds(bi * R, R)]], rows, sem).wait()

            def grp_g(g, _):
                locs = mloc[pl.ds(bi * R + g * L, L)]
                for lane in range(L):
                    loc = locs[lane]
                    j = g * L + lane
                    for k in range(KD):
                        a = acc[loc, pl.ds(k * L, L)]
                        v = rows[j, pl.ds(k * L, L)]
                        acc[loc, pl.ds(k * L, L)] = jnp.maximum(a, v)
                return 0
            lax.fori_loop(0, R // L, grp_g, 0)
            return 0
        lax.fori_loop(0, nb, batch_b, 0)
        return 0

    lax.fori_loop(0, E_HALF // C, chunk_body, 0)

    pltpu.sync_copy(acc.at[pl.ds(0, NPT), :], out_hbm.at[c, pl.ds(lo, NPT), :])


@jax.jit
def _sc_segmax(x, edge_index):
    mesh = plsc.VectorSubcoreMesh(core_axis_name="c", subcore_axis_name="s")
    return pl.kernel(
        _sc_body,
        out_type=jax.ShapeDtypeStruct((NC, NPAD, D), jnp.float32),
        mesh=mesh,
        compiler_params=pltpu.CompilerParams(needs_layout_passes=False),
        scratch_types=[
            pltpu.VMEM((NPT + 1, D), jnp.float32),
            pltpu.VMEM((2, C), jnp.int32),
            pltpu.VMEM((CBUF,), jnp.int32),
            pltpu.VMEM((CBUF,), jnp.int32),
            pltpu.VMEM((R, D), jnp.float32),
            pltpu.SemaphoreType.DMA,
        ],
    )(x, edge_index)


def _tc_body(x_ref, p_ref, w1_ref, w2_ref, b_ref, o_ref):
    m = jnp.maximum(p_ref[0], p_ref[1])
    x = x_ref[...]
    agg = jnp.where(m == -jnp.inf, 0.0, m - x)
    h = (jnp.dot(x, w1_ref[...], preferred_element_type=jnp.float32)
         + jnp.dot(agg, w2_ref[...], preferred_element_type=jnp.float32)
         + b_ref[...])
    o_ref[...] = jnp.maximum(h, 0.0)


@jax.jit
def _tc_mlp(x, part, w1, w2, b2d):
    blk = 2000
    grid = N // blk
    return pl.pallas_call(
        _tc_body,
        grid=(grid,),
        in_specs=[
            pl.BlockSpec((blk, D), lambda i: (i, 0)),
            pl.BlockSpec((NC, blk, D), lambda i: (0, i, 0)),
            pl.BlockSpec((D, D), lambda i: (0, 0)),
            pl.BlockSpec((D, D), lambda i: (0, 0)),
            pl.BlockSpec((1, D), lambda i: (0, 0)),
        ],
        out_specs=pl.BlockSpec((blk, D), lambda i: (i, 0)),
        out_shape=jax.ShapeDtypeStruct((N, D), jnp.float32),
    )(x, part, w1, w2, b2d)


def kernel(x, edge_index, W, b):
    part = _sc_segmax(x, edge_index)
    return _tc_mlp(x, part, W[:D], W[D:], b.reshape(1, D))


# speedup vs baseline: 1.8010x; 1.3615x over previous
import functools

import jax
import jax.numpy as jnp
from jax import lax
from jax.experimental import pallas as pl
from jax.experimental.pallas import tpu as pltpu
from jax.experimental.pallas import tpu_sc as plsc

N = 10000
E = 320000
D = 128
L = 16
NC = 2
NS = 16
NPT = 640
NPAD = NS * NPT
E_HALF = E // NC
C = 3200
NCH = E_HALF // C
R = 64
NBUF = 3
CBUF = ((C + R - 1) // R) * R
KD = D // L


def _sc_body(x_hbm, edge_hbm, out_hbm, acc, eb0, eb1, msrc, mloc, rows,
             esem, gsem):
    c = lax.axis_index("c")
    s = lax.axis_index("s")
    lo = s * NPT
    neg = jnp.full((L,), -jnp.inf, dtype=jnp.float32)
    zero_idx = jnp.zeros((L,), dtype=jnp.int32)
    dummy = jnp.full((L,), NPT, dtype=jnp.int32)
    iot = lax.iota(jnp.int32, L)

    def init_r(r, _):
        for k in range(KD):
            acc[r, pl.ds(k * L, L)] = neg
        return 0
    lax.fori_loop(0, NPT + 1, init_r, 0)

    def init_m(i, _):
        msrc[pl.ds(i * L, L)] = zero_idx
        mloc[pl.ds(i * L, L)] = dummy
        return 0
    lax.fori_loop(0, CBUF // L, init_m, 0)

    ebufs = (eb0, eb1)
    pltpu.make_async_copy(
        edge_hbm.at[:, pl.ds(c * E_HALF, C)], eb0, esem).start()

    lov = jnp.broadcast_to(lo, (L,))
    zv = jnp.zeros((L,), dtype=jnp.int32)
    nptv = jnp.full((L,), NPT, dtype=jnp.int32)
    onev = jnp.full((L,), 1, dtype=jnp.int32)

    def chunk(ch, eb, nxt_eb):
        off = c * E_HALF + ch * C
        pltpu.make_async_copy(
            edge_hbm.at[:, pl.ds(off, C)], eb, esem).wait()

        @pl.when(ch + 1 < NCH)
        def _():
            pltpu.make_async_copy(
                edge_hbm.at[:, pl.ds(off + C, C)], nxt_eb, esem).start()

        def scan_i(i, cntv):
            d = eb[1, pl.ds(i * L, L)]
            sv = eb[0, pl.ds(i * L, L)]
            locv = d - lov
            m = (locv >= zv) & (locv < nptv)
            mi = jnp.where(m, onev, zv)
            pos = cntv + plsc.cumsum(mi) - onev
            plsc.store_scatter(msrc, [pos], sv, mask=m)
            plsc.store_scatter(mloc, [pos], locv, mask=m)
            return cntv + plsc.all_reduce_population_count(m)
        cntv = lax.fori_loop(0, C // L, scan_i, zv)
        cnt = jnp.max(cntv)
        nb = (cnt + R - 1) // R

        base = (cnt // L) * L
        idxv = iot + jnp.broadcast_to(base, (L,))
        mfill = idxv >= jnp.broadcast_to(cnt, (L,))
        plsc.store_scatter(mloc, [idxv], dummy, mask=mfill)

        def fill_g(g, _):
            mloc[pl.ds(g * L, L)] = dummy
            return 0
        lax.fori_loop(base // L + 1, (nb * R) // L, fill_g, 0)

        for b in range(NBUF):
            @pl.when(b < nb)
            def _(b=b):
                pltpu.make_async_copy(
                    x_hbm.at[msrc.at[pl.ds(b * R, R)]], rows.at[b],
                    gsem).start()

        def outer(o, _):
            for b in range(NBUF):
                bi = o * NBUF + b

                @pl.when(bi < nb)
                def _(bi=bi, b=b):
                    pltpu.make_async_copy(
                        x_hbm.at[msrc.at[pl.ds(bi * R, R)]], rows.at[b],
                        gsem).wait()

                    def grp(g, _):
                        locs = mloc[pl.ds(bi * R + g * L, L)]
                        for lane in range(L):
                            loc = locs[lane]
                            j = g * L + lane
                            for k in range(KD):
                                a = acc[loc, pl.ds(k * L, L)]
                                v = rows[b, j, pl.ds(k * L, L)]
                                acc[loc, pl.ds(k * L, L)] = jnp.maximum(a, v)
                        return 0
                    lax.fori_loop(0, R // L, grp, 0)

                    @pl.when(bi + NBUF < nb)
                    def _():
                        pltpu.make_async_copy(
                            x_hbm.at[msrc.at[pl.ds((bi + NBUF) * R, R)]],
                            rows.at[b], gsem).start()
            return 0
        lax.fori_loop(0, (nb + NBUF - 1) // NBUF, outer, 0)

    def pair(p, _):
        chunk(p * 2, ebufs[0], ebufs[1])
        chunk(p * 2 + 1, ebufs[1], ebufs[0])
        return 0
    lax.fori_loop(0, NCH // 2, pair, 0)

    pltpu.sync_copy(acc.at[pl.ds(0, NPT), :], out_hbm.at[c, pl.ds(lo, NPT), :])


@jax.jit
def _sc_segmax(x, edge_index):
    mesh = plsc.VectorSubcoreMesh(core_axis_name="c", subcore_axis_name="s")
    return pl.kernel(
        _sc_body,
        out_type=jax.ShapeDtypeStruct((NC, NPAD, D), jnp.float32),
        mesh=mesh,
        compiler_params=pltpu.CompilerParams(needs_layout_passes=False),
        scratch_types=[
            pltpu.VMEM((NPT + 1, D), jnp.float32),
            pltpu.VMEM((2, C), jnp.int32),
            pltpu.VMEM((2, C), jnp.int32),
            pltpu.VMEM((CBUF,), jnp.int32),
            pltpu.VMEM((CBUF,), jnp.int32),
            pltpu.VMEM((NBUF, R, D), jnp.float32),
            pltpu.SemaphoreType.DMA,
            pltpu.SemaphoreType.DMA,
        ],
    )(x, edge_index)


def _tc_body(x_ref, p_ref, w1_ref, w2_ref, b_ref, o_ref):
    m = jnp.maximum(p_ref[0], p_ref[1])
    x = x_ref[...]
    agg = jnp.where(m == -jnp.inf, 0.0, m - x)
    h = (jnp.dot(x, w1_ref[...], preferred_element_type=jnp.float32)
         + jnp.dot(agg, w2_ref[...], preferred_element_type=jnp.float32)
         + b_ref[...])
    o_ref[...] = jnp.maximum(h, 0.0)


@jax.jit
def _tc_mlp(x, part, w1, w2, b2d):
    blk = 2000
    grid = N // blk
    return pl.pallas_call(
        _tc_body,
        grid=(grid,),
        in_specs=[
            pl.BlockSpec((blk, D), lambda i: (i, 0)),
            pl.BlockSpec((NC, blk, D), lambda i: (0, i, 0)),
            pl.BlockSpec((D, D), lambda i: (0, 0)),
            pl.BlockSpec((D, D), lambda i: (0, 0)),
            pl.BlockSpec((1, D), lambda i: (0, 0)),
        ],
        out_specs=pl.BlockSpec((blk, D), lambda i: (i, 0)),
        out_shape=jax.ShapeDtypeStruct((N, D), jnp.float32),
    )(x, part, w1, w2, b2d)


def kernel(x, edge_index, W, b):
    part = _sc_segmax(x, edge_index)
    return _tc_mlp(x, part, W[:D], W[D:], b.reshape(1, D))


# speedup vs baseline: 1.8380x; 1.0205x over previous
import functools

import jax
import jax.numpy as jnp
from jax import lax
from jax.experimental import pallas as pl
from jax.experimental.pallas import tpu as pltpu
from jax.experimental.pallas import tpu_sc as plsc

N = 10000
E = 320000
D = 128
L = 16
NC = 2
NS = 16
NPT = 640
NPAD = NS * NPT
E_HALF = E // NC
C = 3200
NCH = E_HALF // C
R = 64
NBUF = 3
CBUF = ((C + R - 1) // R) * R + L
KD = D // L


def _sc_body(x_hbm, edge_hbm, out_hbm, acc, eb0, eb1, msrc, mloc, rows,
             esem, gsem):
    c = lax.axis_index("c")
    s = lax.axis_index("s")
    lo = s * NPT
    neg = jnp.full((L,), -jnp.inf, dtype=jnp.float32)
    zero_idx = jnp.zeros((L,), dtype=jnp.int32)
    dummy = jnp.full((L,), NPT, dtype=jnp.int32)
    iot = lax.iota(jnp.int32, L)

    def init_r(r, _):
        for k in range(KD):
            acc[r, pl.ds(k * L, L)] = neg
        return 0
    lax.fori_loop(0, NPT + 1, init_r, 0)

    def init_m(i, _):
        msrc[pl.ds(i * L, L)] = zero_idx
        mloc[pl.ds(i * L, L)] = dummy
        return 0
    lax.fori_loop(0, CBUF // L, init_m, 0)

    ebufs = (eb0, eb1)
    pltpu.make_async_copy(
        edge_hbm.at[:, pl.ds(c * E_HALF, C)], eb0, esem).start()

    lov = jnp.broadcast_to(lo, (L,))
    nptu = jnp.full((L,), NPT, dtype=jnp.uint32)

    def chunk(ch, eb, nxt_eb):
        off = c * E_HALF + ch * C
        pltpu.make_async_copy(
            edge_hbm.at[:, pl.ds(off, C)], eb, esem).wait()

        @pl.when(ch + 1 < NCH)
        def _():
            pltpu.make_async_copy(
                edge_hbm.at[:, pl.ds(off + C, C)], nxt_eb, esem).start()

        def scan_i(i, cnt):
            d = eb[1, pl.ds(i * L, L)]
            sv = eb[0, pl.ds(i * L, L)]
            locv = d - lov
            m = plsc.bitcast(locv, jnp.uint32) < nptu
            plsc.store_compressed(msrc.at[pl.ds(cnt, L)], sv, mask=m)
            plsc.store_compressed(mloc.at[pl.ds(cnt, L)], locv, mask=m)
            return cnt + plsc.all_reduce_population_count(m)[0]
        cnt = lax.fori_loop(0, C // L, scan_i, jnp.int32(0))
        nb = (cnt + R - 1) // R

        base = (cnt // L) * L
        idxv = iot + jnp.broadcast_to(base, (L,))
        mfill = idxv >= jnp.broadcast_to(cnt, (L,))
        plsc.store_scatter(mloc, [idxv], dummy, mask=mfill)

        def fill_g(g, _):
            mloc[pl.ds(g * L, L)] = dummy
            return 0
        lax.fori_loop(base // L + 1, (nb * R) // L, fill_g, 0)

        for b in range(NBUF):
            @pl.when(b < nb)
            def _(b=b):
                pltpu.make_async_copy(
                    x_hbm.at[msrc.at[pl.ds(b * R, R)]], rows.at[b],
                    gsem).start()

        def outer(o, _):
            for b in range(NBUF):
                bi = o * NBUF + b

                @pl.when(bi < nb)
                def _(bi=bi, b=b):
                    pltpu.make_async_copy(
                        x_hbm.at[msrc.at[pl.ds(bi * R, R)]], rows.at[b],
                        gsem).wait()

                    def grp(g, _):
                        locs = mloc[pl.ds(bi * R + g * L, L)]
                        locs_s = [locs[lane] for lane in range(L)]
                        for lane in range(L):
                            loc = locs_s[lane]
                            j = g * L + lane
                            avs = [acc[loc, pl.ds(k * L, L)]
                                   for k in range(KD)]
                            vvs = [rows[b, j, pl.ds(k * L, L)]
                                   for k in range(KD)]
                            for k in range(KD):
                                acc[loc, pl.ds(k * L, L)] = jnp.maximum(
                                    avs[k], vvs[k])
                        return 0
                    lax.fori_loop(0, R // L, grp, 0)

                    @pl.when(bi + NBUF < nb)
                    def _():
                        pltpu.make_async_copy(
                            x_hbm.at[msrc.at[pl.ds((bi + NBUF) * R, R)]],
                            rows.at[b], gsem).start()
            return 0
        lax.fori_loop(0, (nb + NBUF - 1) // NBUF, outer, 0)

    def pair(p, _):
        chunk(p * 2, ebufs[0], ebufs[1])
        chunk(p * 2 + 1, ebufs[1], ebufs[0])
        return 0
    lax.fori_loop(0, NCH // 2, pair, 0)

    pltpu.sync_copy(acc.at[pl.ds(0, NPT), :], out_hbm.at[c, pl.ds(lo, NPT), :])


@jax.jit
def _sc_segmax(x, edge_index):
    mesh = plsc.VectorSubcoreMesh(core_axis_name="c", subcore_axis_name="s")
    return pl.kernel(
        _sc_body,
        out_type=jax.ShapeDtypeStruct((NC, NPAD, D), jnp.float32),
        mesh=mesh,
        compiler_params=pltpu.CompilerParams(needs_layout_passes=False),
        scratch_types=[
            pltpu.VMEM((NPT + 1, D), jnp.float32),
            pltpu.VMEM((2, C), jnp.int32),
            pltpu.VMEM((2, C), jnp.int32),
            pltpu.VMEM((CBUF,), jnp.int32),
            pltpu.VMEM((CBUF,), jnp.int32),
            pltpu.VMEM((NBUF, R, D), jnp.float32),
            pltpu.SemaphoreType.DMA,
            pltpu.SemaphoreType.DMA,
        ],
    )(x, edge_index)


def _tc_body(x_ref, p_ref, w1_ref, w2_ref, b_ref, o_ref):
    m = jnp.maximum(p_ref[0], p_ref[1])
    x = x_ref[...]
    agg = jnp.where(m == -jnp.inf, 0.0, m - x)
    h = (jnp.dot(x, w1_ref[...], preferred_element_type=jnp.float32)
         + jnp.dot(agg, w2_ref[...], preferred_element_type=jnp.float32)
         + b_ref[...])
    o_ref[...] = jnp.maximum(h, 0.0)


@jax.jit
def _tc_mlp(x, part, w1, w2, b2d):
    blk = 2000
    grid = N // blk
    return pl.pallas_call(
        _tc_body,
        grid=(grid,),
        in_specs=[
            pl.BlockSpec((blk, D), lambda i: (i, 0)),
            pl.BlockSpec((NC, blk, D), lambda i: (0, i, 0)),
            pl.BlockSpec((D, D), lambda i: (0, 0)),
            pl.BlockSpec((D, D), lambda i: (0, 0)),
            pl.BlockSpec((1, D), lambda i: (0, 0)),
        ],
        out_specs=pl.BlockSpec((blk, D), lambda i: (i, 0)),
        out_shape=jax.ShapeDtypeStruct((N, D), jnp.float32),
    )(x, part, w1, w2, b2d)


def kernel(x, edge_index, W, b):
    part = _sc_segmax(x, edge_index)
    return _tc_mlp(x, part, W[:D], W[D:], b.reshape(1, D))
